# Initial kernel scaffold; baseline (speedup 1.0000x reference)
import jax
import jax.numpy as jnp
from jax.experimental import pallas as pl


def kernel(edge_index, num_nodes):
    raise NotImplementedError("write your pallas kernel here")


# speedup vs baseline: 25.9967x; 25.9967x over previous
import functools

import jax
import jax.numpy as jnp
from jax import lax
from jax.experimental import pallas as pl
from jax.experimental.pallas import tpu as pltpu
from jax.experimental.pallas import tpu_sc as plsc

_N_NODES = 100000
_NC, _NS = 2, 16
_NW = _NC * _NS
_CHUNK = 80
_ROWS = 250
_FETCHES = 10
_HIST_PAD = 100352
_ZCHUNK = _HIST_PAD // _NS


def _sc_histogram(src):
    mesh = plsc.VectorSubcoreMesh(core_axis_name="c", subcore_axis_name="s")

    @functools.partial(
        pl.kernel,
        mesh=mesh,
        out_type=jax.ShapeDtypeStruct((_NC, _N_NODES), jnp.float32),
        scratch_types=[
            pltpu.VMEM((_ROWS, _CHUNK), jnp.int32),
            pltpu.VMEM((_CHUNK,), jnp.float32),
            pltpu.VMEM((_ZCHUNK,), jnp.float32),
            pltpu.VMEM_SHARED((_HIST_PAD,), jnp.float32),
        ],
        compiler_params=pltpu.CompilerParams(use_tc_tiling_on_sc=False),
    )
    def hist_kernel(src_hbm, out_hbm, idx_v, ones_v, zeros_v, hist_sh):
        c = lax.axis_index("c")
        s = lax.axis_index("s")
        wid = c * _NS + s

        ones16 = jnp.ones((16,), jnp.float32)
        for i in range(_CHUNK // 16):
            ones_v[pl.ds(i * 16, 16)] = ones16

        zeros16 = jnp.zeros((16,), jnp.float32)

        def zfill(i, carry):
            zeros_v[pl.ds(pl.multiple_of(i * 16, 16), 16)] = zeros16
            return carry

        lax.fori_loop(0, _ZCHUNK // 16, zfill, 0)

        pltpu.sync_copy(
            zeros_v, hist_sh.at[pl.ds(pl.multiple_of(s * _ZCHUNK, 8), _ZCHUNK)]
        )
        plsc.subcore_barrier()

        def fetch(f, carry):
            pltpu.sync_copy(src_hbm.at[wid, f], idx_v)

            def row(j, c2):
                pltpu.sync_copy(ones_v, hist_sh.at[idx_v.at[j]], add=True)
                return c2

            lax.fori_loop(0, _ROWS, row, 0)
            return carry

        lax.fori_loop(0, _FETCHES, fetch, 0)

        plsc.subcore_barrier()

        @pl.when(s == 0)
        def _():
            pltpu.sync_copy(hist_sh.at[pl.ds(0, _N_NODES)], out_hbm.at[c])

    return hist_kernel(src)


def _combine(n_arr, partials):
    def body(n_ref, p_ref, o_ref):
        tot = p_ref[0:1, :] + p_ref[1:2, :]
        iota = lax.broadcasted_iota(jnp.int32, (1, _N_NODES), 1)
        o_ref[...] = jnp.where(iota < n_ref[0], tot, 0.0)

    return pl.pallas_call(
        body,
        out_shape=jax.ShapeDtypeStruct((1, _N_NODES), jnp.float32),
        in_specs=[
            pl.BlockSpec(memory_space=pltpu.SMEM),
            pl.BlockSpec(memory_space=pltpu.VMEM),
        ],
        out_specs=pl.BlockSpec(memory_space=pltpu.VMEM),
    )(n_arr, partials)


def kernel(edge_index, num_nodes):
    src = edge_index[0].astype(jnp.int32).reshape(_NW, _FETCHES, _ROWS, _CHUNK)
    partials = _sc_histogram(src)
    n_arr = jnp.asarray(num_nodes, jnp.int32).reshape(1)
    deg = _combine(n_arr, partials)
    return deg.reshape(_N_NODES, 1)


# speedup vs baseline: 39.0520x; 1.5022x over previous
import functools

import jax
import jax.numpy as jnp
from jax import lax
from jax.experimental import pallas as pl
from jax.experimental.pallas import tpu as pltpu
from jax.experimental.pallas import tpu_sc as plsc

_N_NODES = 100000
_NC, _NS = 2, 16
_NW = _NC * _NS
_FETCH = 20000
_FETCHES = 10
_GROUPS = _FETCH // 16
_UNROLL = 10


def _sc_histogram(src):
    mesh = plsc.VectorSubcoreMesh(core_axis_name="c", subcore_axis_name="s")

    @functools.partial(
        pl.kernel,
        mesh=mesh,
        out_type=jax.ShapeDtypeStruct((_NW, _N_NODES), jnp.float32),
        scratch_types=[
            pltpu.VMEM((_FETCH,), jnp.int32),
            pltpu.VMEM((_N_NODES,), jnp.float32),
        ],
        compiler_params=pltpu.CompilerParams(
            use_tc_tiling_on_sc=False, needs_layout_passes=False
        ),
    )
    def hist_kernel(src_hbm, out_hbm, idx_v, hist_v):
        c = lax.axis_index("c")
        s = lax.axis_index("s")
        wid = c * _NS + s

        zeros16 = jnp.zeros((16,), jnp.float32)
        ones16 = jnp.ones((16,), jnp.float32)

        def zfill(i, carry):
            hist_v[pl.ds(pl.multiple_of(i * 16, 16), 16)] = zeros16
            return carry

        lax.fori_loop(0, _N_NODES // 16, zfill, 0)

        def fetch(f, carry):
            pltpu.sync_copy(src_hbm.at[wid, f], idx_v)

            def group(g, c2):
                for u in range(_UNROLL):
                    base = pl.multiple_of((g * _UNROLL + u) * 16, 16)
                    idx = idx_v[pl.ds(base, 16)]
                    plsc.addupdate_scatter(hist_v, [idx], ones16)
                return c2

            lax.fori_loop(0, _GROUPS // _UNROLL, group, 0)
            return carry

        lax.fori_loop(0, _FETCHES, fetch, 0)

        pltpu.sync_copy(hist_v, out_hbm.at[wid])

    return hist_kernel(src)


def _combine(n_arr, partials):
    def body(n_ref, p_ref, o_ref):
        tot = jnp.sum(p_ref[...], axis=0, keepdims=True)
        iota = lax.broadcasted_iota(jnp.int32, (1, _N_NODES), 1)
        o_ref[...] = jnp.where(iota < n_ref[0], tot, 0.0)

    return pl.pallas_call(
        body,
        out_shape=jax.ShapeDtypeStruct((1, _N_NODES), jnp.float32),
        in_specs=[
            pl.BlockSpec(memory_space=pltpu.SMEM),
            pl.BlockSpec(memory_space=pltpu.VMEM),
        ],
        out_specs=pl.BlockSpec(memory_space=pltpu.VMEM),
    )(n_arr, partials)


def kernel(edge_index, num_nodes):
    src = edge_index[0].astype(jnp.int32).reshape(_NW, _FETCHES, _FETCH)
    partials = _sc_histogram(src)
    n_arr = jnp.asarray(num_nodes, jnp.int32).reshape(1)
    deg = _combine(n_arr, partials)
    return deg.reshape(_N_NODES, 1)


# speedup vs baseline: 43.6804x; 1.1185x over previous
import functools

import jax
import jax.numpy as jnp
from jax import lax
from jax.experimental import pallas as pl
from jax.experimental.pallas import tpu as pltpu
from jax.experimental.pallas import tpu_sc as plsc

_N_NODES = 100000
_NC, _NS = 2, 16
_NW = _NC * _NS
_FETCH = 8000
_FETCHES = 25
_GROUPS = _FETCH // 16
_UNROLL = 10


def _sc_histogram(edges):
    mesh = plsc.VectorSubcoreMesh(core_axis_name="c", subcore_axis_name="s")

    @functools.partial(
        pl.kernel,
        mesh=mesh,
        out_type=jax.ShapeDtypeStruct((_NW, _N_NODES), jnp.float32),
        scratch_types=[
            pltpu.VMEM((2, _FETCH), jnp.int32),
            pltpu.VMEM((_N_NODES,), jnp.float32),
            pltpu.SemaphoreType.DMA,
        ],
        compiler_params=pltpu.CompilerParams(
            use_tc_tiling_on_sc=False, needs_layout_passes=False
        ),
    )
    def hist_kernel(src_hbm, out_hbm, idx_v, hist_v, sem):
        c = lax.axis_index("c")
        s = lax.axis_index("s")
        wid = c * _NS + s

        zeros16 = jnp.zeros((16,), jnp.float32)
        ones16 = jnp.ones((16,), jnp.float32)

        def zfill(i, carry):
            for u in range(_UNROLL):
                base = pl.multiple_of((i * _UNROLL + u) * 16, 16)
                hist_v[pl.ds(base, 16)] = zeros16
            return carry

        lax.fori_loop(0, _N_NODES // 16 // _UNROLL, zfill, 0)

        def start(f):
            return pltpu.async_copy(src_hbm.at[0, wid, f], idx_v.at[f % 2], sem)

        cp = start(0)
        for f in range(_FETCHES):
            cp.wait()
            if f + 1 < _FETCHES:
                cp = start(f + 1)
            buf = idx_v.at[f % 2]

            def group(g, c2):
                for u in range(_UNROLL):
                    base = pl.multiple_of((g * _UNROLL + u) * 16, 16)
                    idx = buf[pl.ds(base, 16)]
                    plsc.addupdate_scatter(hist_v, [idx], ones16)
                return c2

            lax.fori_loop(0, _GROUPS // _UNROLL, group, 0)

        pltpu.sync_copy(hist_v, out_hbm.at[wid])

    return hist_kernel(edges)


def _combine(n_arr, partials):
    def body(n_ref, p_ref, o_ref):
        tot = jnp.sum(p_ref[...], axis=0, keepdims=True)
        iota = lax.broadcasted_iota(jnp.int32, (1, _N_NODES), 1)
        o_ref[...] = jnp.where(iota < n_ref[0], tot, 0.0)

    return pl.pallas_call(
        body,
        out_shape=jax.ShapeDtypeStruct((1, _N_NODES), jnp.float32),
        in_specs=[
            pl.BlockSpec(memory_space=pltpu.SMEM),
            pl.BlockSpec(memory_space=pltpu.VMEM),
        ],
        out_specs=pl.BlockSpec(memory_space=pltpu.VMEM),
    )(n_arr, partials)


def kernel(edge_index, num_nodes):
    edges = edge_index.astype(jnp.int32).reshape(2, _NW, _FETCHES, _FETCH)
    partials = _sc_histogram(edges)
    n_arr = jnp.asarray(num_nodes, jnp.int32).reshape(1)
    deg = _combine(n_arr, partials)
    return deg.reshape(_N_NODES, 1)


# speedup vs baseline: 43.7386x; 1.0013x over previous
import functools

import jax
import jax.numpy as jnp
from jax import lax
from jax.experimental import pallas as pl
from jax.experimental.pallas import tpu as pltpu
from jax.experimental.pallas import tpu_sc as plsc

_N_NODES = 100000
_NC, _NS = 2, 16
_NW = _NC * _NS
_FETCH = 8000
_FETCHES = 25
_GROUPS = _FETCH // 16
_UNROLL = 10


def _sc_histogram(edges):
    mesh = plsc.VectorSubcoreMesh(core_axis_name="c", subcore_axis_name="s")

    @functools.partial(
        pl.kernel,
        mesh=mesh,
        out_type=jax.ShapeDtypeStruct((_NW, _N_NODES), jnp.float32),
        scratch_types=[
            pltpu.VMEM((2, _FETCH), jnp.int32),
            pltpu.VMEM((_N_NODES,), jnp.float32),
            pltpu.SemaphoreType.DMA,
        ],
        compiler_params=pltpu.CompilerParams(
            use_tc_tiling_on_sc=False, needs_layout_passes=False
        ),
    )
    def hist_kernel(src_hbm, out_hbm, idx_v, hist_v, sem):
        c = lax.axis_index("c")
        s = lax.axis_index("s")
        wid = c * _NS + s

        zeros16 = jnp.zeros((16,), jnp.float32)
        ones16 = jnp.ones((16,), jnp.float32)

        def zfill(i, carry):
            for u in range(_UNROLL):
                base = pl.multiple_of((i * _UNROLL + u) * 16, 16)
                hist_v[pl.ds(base, 16)] = zeros16
            return carry

        lax.fori_loop(0, _N_NODES // 16 // _UNROLL, zfill, 0)

        tile_base = wid * (_FETCHES * _FETCH)

        def start(f):
            off = pl.multiple_of(tile_base + f * _FETCH, 8)
            return pltpu.async_copy(
                src_hbm.at[0, pl.ds(off, _FETCH)], idx_v.at[f % 2], sem
            )

        cp = start(0)
        for f in range(_FETCHES):
            cp.wait()
            if f + 1 < _FETCHES:
                cp = start(f + 1)
            buf = idx_v.at[f % 2]

            def group(g, c2):
                for u in range(_UNROLL):
                    base = pl.multiple_of((g * _UNROLL + u) * 16, 16)
                    idx = buf[pl.ds(base, 16)]
                    plsc.addupdate_scatter(hist_v, [idx], ones16)
                return c2

            lax.fori_loop(0, _GROUPS // _UNROLL, group, 0)

        pltpu.sync_copy(hist_v, out_hbm.at[wid])

    return hist_kernel(edges)


def _combine(n_arr, partials):
    def body(n_ref, p_ref, o_ref):
        tot = jnp.sum(p_ref[...], axis=0, keepdims=True)
        iota = lax.broadcasted_iota(jnp.int32, (1, _N_NODES), 1)
        o_ref[...] = jnp.where(iota < n_ref[0], tot, 0.0)

    return pl.pallas_call(
        body,
        out_shape=jax.ShapeDtypeStruct((1, _N_NODES), jnp.float32),
        in_specs=[
            pl.BlockSpec(memory_space=pltpu.SMEM),
            pl.BlockSpec(memory_space=pltpu.VMEM),
        ],
        out_specs=pl.BlockSpec(memory_space=pltpu.VMEM),
    )(n_arr, partials)


def kernel(edge_index, num_nodes):
    partials = _sc_histogram(edge_index.astype(jnp.int32))
    n_arr = jnp.asarray(num_nodes, jnp.int32).reshape(1)
    deg = _combine(n_arr, partials)
    return deg.reshape(_N_NODES, 1)


# speedup vs baseline: 48.9299x; 1.1187x over previous
import functools

import jax
import jax.numpy as jnp
from jax import lax
from jax.experimental import pallas as pl
from jax.experimental.pallas import tpu as pltpu
from jax.experimental.pallas import tpu_sc as plsc

_N_NODES = 100000
_NC, _NS = 2, 16
_NW = _NC * _NS
_FETCH = 8000
_FETCHES = 25
_GROUPS = _FETCH // 16
_UNROLL = 10


def _sc_histogram(src):
    mesh = plsc.VectorSubcoreMesh(core_axis_name="c", subcore_axis_name="s")

    @functools.partial(
        pl.kernel,
        mesh=mesh,
        out_type=jax.ShapeDtypeStruct((_NW, _N_NODES), jnp.float32),
        scratch_types=[
            pltpu.VMEM((2, _FETCH), jnp.int32),
            pltpu.VMEM((_N_NODES,), jnp.float32),
            pltpu.SemaphoreType.DMA,
        ],
        compiler_params=pltpu.CompilerParams(
            use_tc_tiling_on_sc=False, needs_layout_passes=False
        ),
    )
    def hist_kernel(src_hbm, out_hbm, idx_v, hist_v, sem):
        c = lax.axis_index("c")
        s = lax.axis_index("s")
        wid = c * _NS + s

        zeros16 = jnp.zeros((16,), jnp.float32)
        ones16 = jnp.ones((16,), jnp.float32)

        def zfill(i, carry):
            for u in range(_UNROLL):
                base = pl.multiple_of((i * _UNROLL + u) * 16, 16)
                hist_v[pl.ds(base, 16)] = zeros16
            return carry

        lax.fori_loop(0, _N_NODES // 16 // _UNROLL, zfill, 0)

        tile_base = wid * (_FETCHES * _FETCH)

        def start(f):
            off = pl.multiple_of(tile_base + f * _FETCH, 8)
            return pltpu.async_copy(
                src_hbm.at[pl.ds(off, _FETCH)], idx_v.at[f % 2], sem
            )

        cp = start(0)
        for f in range(_FETCHES):
            cp.wait()
            if f + 1 < _FETCHES:
                cp = start(f + 1)
            buf = idx_v.at[f % 2]

            def group(g, c2):
                for u in range(_UNROLL):
                    base = pl.multiple_of((g * _UNROLL + u) * 16, 16)
                    idx = buf[pl.ds(base, 16)]
                    plsc.addupdate_scatter(hist_v, [idx], ones16)
                return c2

            lax.fori_loop(0, _GROUPS // _UNROLL, group, 0)

        pltpu.sync_copy(hist_v, out_hbm.at[wid])

    return hist_kernel(src)


def _combine(n_arr, partials):
    def body(n_ref, p_ref, o_ref):
        tot = jnp.sum(p_ref[...], axis=0, keepdims=True)
        iota = lax.broadcasted_iota(jnp.int32, (1, _N_NODES), 1)
        o_ref[...] = jnp.where(iota < n_ref[0], tot, 0.0)

    return pl.pallas_call(
        body,
        out_shape=jax.ShapeDtypeStruct((1, _N_NODES), jnp.float32),
        in_specs=[
            pl.BlockSpec(memory_space=pltpu.SMEM),
            pl.BlockSpec(memory_space=pltpu.VMEM),
        ],
        out_specs=pl.BlockSpec(memory_space=pltpu.VMEM),
    )(n_arr, partials)


def kernel(edge_index, num_nodes):
    partials = _sc_histogram(edge_index[0].astype(jnp.int32))
    n_arr = jnp.asarray(num_nodes, jnp.int32).reshape(1)
    deg = _combine(n_arr, partials)
    return deg.reshape(_N_NODES, 1)


# speedup vs baseline: 51.7119x; 1.0569x over previous
import functools

import jax
import jax.numpy as jnp
from jax import lax
from jax.experimental import pallas as pl
from jax.experimental.pallas import tpu as pltpu
from jax.experimental.pallas import tpu_sc as plsc

_N_NODES = 100000
_NC, _NS = 2, 16
_NW = _NC * _NS
_FETCH = 8000
_FETCHES = 25
_GROUPS = _FETCH // 16
_UNROLL = 10
_N_PAD = 100352
_STRIP = _N_PAD // _NW

_mesh = plsc.VectorSubcoreMesh(core_axis_name="c", subcore_axis_name="s")
_params = pltpu.CompilerParams(use_tc_tiling_on_sc=False, needs_layout_passes=False)


def _sc_histogram(src):
    @functools.partial(
        pl.kernel,
        mesh=_mesh,
        out_type=jax.ShapeDtypeStruct((_NW, _N_PAD), jnp.float32),
        scratch_types=[
            pltpu.VMEM((2, _FETCH), jnp.int32),
            pltpu.VMEM((_N_PAD,), jnp.float32),
            pltpu.SemaphoreType.DMA,
        ],
        compiler_params=_params,
    )
    def hist_kernel(src_hbm, out_hbm, idx_v, hist_v, sem):
        c = lax.axis_index("c")
        s = lax.axis_index("s")
        wid = c * _NS + s

        zeros16 = jnp.zeros((16,), jnp.float32)
        ones16 = jnp.ones((16,), jnp.float32)

        def zfill(i, carry):
            for u in range(16):
                base = pl.multiple_of((i * 16 + u) * 16, 16)
                hist_v[pl.ds(base, 16)] = zeros16
            return carry

        lax.fori_loop(0, _N_PAD // 256, zfill, 0)

        tile_base = wid * (_FETCHES * _FETCH)

        def start(f):
            off = pl.multiple_of(tile_base + f * _FETCH, 8)
            return pltpu.async_copy(
                src_hbm.at[pl.ds(off, _FETCH)], idx_v.at[f % 2], sem
            )

        cp = start(0)
        for f in range(_FETCHES):
            cp.wait()
            if f + 1 < _FETCHES:
                cp = start(f + 1)
            buf = idx_v.at[f % 2]

            def group(g, c2):
                for u in range(_UNROLL):
                    base = pl.multiple_of((g * _UNROLL + u) * 16, 16)
                    idx = buf[pl.ds(base, 16)]
                    plsc.addupdate_scatter(hist_v, [idx], ones16)
                return c2

            lax.fori_loop(0, _GROUPS // _UNROLL, group, 0)

        pltpu.sync_copy(hist_v, out_hbm.at[wid])

    return hist_kernel(src)


def _sc_combine(partials, n_arr):
    @functools.partial(
        pl.kernel,
        mesh=_mesh,
        out_type=jax.ShapeDtypeStruct((_N_PAD,), jnp.float32),
        scratch_types=[
            pltpu.VMEM((_NW, _STRIP), jnp.float32),
            pltpu.VMEM((_STRIP,), jnp.float32),
            pltpu.VMEM((16,), jnp.int32),
            pltpu.SemaphoreType.DMA,
        ],
        compiler_params=_params,
    )
    def combine_kernel(parts_hbm, n_hbm, out_hbm, stage_v, acc_v, n_v, sem):
        c = lax.axis_index("c")
        s = lax.axis_index("s")
        wid = c * _NS + s
        base = pl.multiple_of(wid * _STRIP, 8)

        pltpu.sync_copy(n_hbm, n_v)
        cps = [
            pltpu.async_copy(
                parts_hbm.at[r, pl.ds(base, _STRIP)], stage_v.at[r], sem
            )
            for r in range(_NW)
        ]
        for cp in cps:
            cp.wait()

        n_vec = n_v[pl.ds(0, 16)]
        lanes = lax.iota(jnp.int32, 16)

        def group(g, carry):
            for u in range(7):
                off = pl.multiple_of((g * 7 + u) * 16, 16)
                acc = stage_v[0, pl.ds(off, 16)]
                for r in range(1, _NW):
                    acc = acc + stage_v[r, pl.ds(off, 16)]
                bin_ix = (wid * _STRIP + off) + lanes
                acc_v[pl.ds(off, 16)] = jnp.where(bin_ix < n_vec, acc, 0.0)
            return carry

        lax.fori_loop(0, _STRIP // 112, group, 0)

        pltpu.sync_copy(acc_v, out_hbm.at[pl.ds(base, _STRIP)])

    return combine_kernel(partials, n_arr)


def kernel(edge_index, num_nodes):
    src = edge_index[0].astype(jnp.int32)
    partials = _sc_histogram(src)
    n_arr = jnp.full((16,), num_nodes, jnp.int32)
    deg = _sc_combine(partials, n_arr)
    return deg[:_N_NODES].reshape(_N_NODES, 1)


# speedup vs baseline: 79.5294x; 1.5379x over previous
import functools

import jax
import jax.numpy as jnp
from jax import lax
from jax.experimental import pallas as pl
from jax.experimental.pallas import tpu as pltpu
from jax.experimental.pallas import tpu_sc as plsc

_N_NODES = 100000
_NC, _NS = 2, 16
_NW = _NC * _NS
_FETCH = 8000
_FETCHES = 25
_GROUPS = _FETCH // 16
_UNROLL = 10
_N_PAD = 100352
_STRIP = _N_PAD // _NW

_mesh = plsc.VectorSubcoreMesh(core_axis_name="c", subcore_axis_name="s")
_params = pltpu.CompilerParams(use_tc_tiling_on_sc=False, needs_layout_passes=False)


def _sc_histogram(src):
    @functools.partial(
        pl.kernel,
        mesh=_mesh,
        out_type=jax.ShapeDtypeStruct((_NW, _N_PAD), jnp.float32),
        scratch_types=[
            pltpu.VMEM((2, _FETCH), jnp.int32),
            pltpu.VMEM((_N_PAD,), jnp.float32),
            pltpu.SemaphoreType.DMA,
        ],
        compiler_params=_params,
    )
    def hist_kernel(src_hbm, out_hbm, idx_v, hist_v, sem):
        c = lax.axis_index("c")
        s = lax.axis_index("s")
        wid = c * _NS + s

        zeros16 = jnp.zeros((16,), jnp.float32)
        ones16 = jnp.ones((16,), jnp.float32)

        @plsc.parallel_loop(0, _N_PAD // 16, unroll=16)
        def zfill(i):
            hist_v[pl.ds(pl.multiple_of(i * 16, 16), 16)] = zeros16

        tile_base = wid * (_FETCHES * _FETCH)

        def start(f):
            off = pl.multiple_of(tile_base + f * _FETCH, 8)
            return pltpu.async_copy(
                src_hbm.at[pl.ds(off, _FETCH)], idx_v.at[f % 2], sem
            )

        cp = start(0)
        for f in range(_FETCHES):
            cp.wait()
            if f + 1 < _FETCHES:
                cp = start(f + 1)
            buf = idx_v.at[f % 2]

            @plsc.parallel_loop(0, _GROUPS, unroll=_UNROLL)
            def group(g):
                idx = buf[pl.ds(pl.multiple_of(g * 16, 16), 16)]
                plsc.addupdate_scatter(hist_v, [idx], ones16)

        pltpu.sync_copy(hist_v, out_hbm.at[wid])

    return hist_kernel(src)


def _sc_combine(partials, n_arr):
    @functools.partial(
        pl.kernel,
        mesh=_mesh,
        out_type=jax.ShapeDtypeStruct((_N_PAD,), jnp.float32),
        scratch_types=[
            pltpu.VMEM((_NW, _STRIP), jnp.float32),
            pltpu.VMEM((_STRIP,), jnp.float32),
            pltpu.VMEM((16,), jnp.int32),
            pltpu.SemaphoreType.DMA,
        ],
        compiler_params=_params,
    )
    def combine_kernel(parts_hbm, n_hbm, out_hbm, stage_v, acc_v, n_v, sem):
        c = lax.axis_index("c")
        s = lax.axis_index("s")
        wid = c * _NS + s
        base = pl.multiple_of(wid * _STRIP, 8)

        pltpu.sync_copy(n_hbm, n_v)
        cps = [
            pltpu.async_copy(
                parts_hbm.at[r, pl.ds(base, _STRIP)], stage_v.at[r], sem
            )
            for r in range(_NW)
        ]
        for cp in cps:
            cp.wait()

        n_vec = n_v[pl.ds(0, 16)]
        lanes = lax.iota(jnp.int32, 16)

        @plsc.parallel_loop(0, _STRIP // 16, unroll=7)
        def group(g):
            off = pl.multiple_of(g * 16, 16)
            acc = stage_v[0, pl.ds(off, 16)]
            for r in range(1, _NW):
                acc = acc + stage_v[r, pl.ds(off, 16)]
            bin_ix = (wid * _STRIP + off) + lanes
            acc_v[pl.ds(off, 16)] = jnp.where(bin_ix < n_vec, acc, 0.0)

        pltpu.sync_copy(acc_v, out_hbm.at[pl.ds(base, _STRIP)])

    return combine_kernel(partials, n_arr)


def kernel(edge_index, num_nodes):
    src = edge_index[0].astype(jnp.int32)
    partials = _sc_histogram(src)
    n_arr = jnp.full((16,), num_nodes, jnp.int32)
    deg = _sc_combine(partials, n_arr)
    return deg[:_N_NODES].reshape(_N_NODES, 1)


# speedup vs baseline: 95.5891x; 1.2019x over previous
import functools

import jax
import jax.numpy as jnp
from jax import lax
from jax.experimental import pallas as pl
from jax.experimental.pallas import tpu as pltpu
from jax.experimental.pallas import tpu_sc as plsc

_N_NODES = 100000
_NC, _NS = 2, 16
_NW = _NC * _NS
_NBLOCKS = 50000
_BLK = 1562
_REM = _NBLOCKS - _NW * _BLK
_FBLK = 71
_FETCHES = 22
_UNROLL = 2
_N_PAD = 100352
_STRIP = _N_PAD // _NW

_mesh = plsc.VectorSubcoreMesh(core_axis_name="c", subcore_axis_name="s")
_params = pltpu.CompilerParams(use_tc_tiling_on_sc=False, needs_layout_passes=False)


def _sc_histogram(src):
    @functools.partial(
        pl.kernel,
        mesh=_mesh,
        out_type=jax.ShapeDtypeStruct((_NW, _N_PAD), jnp.float32),
        scratch_types=[
            pltpu.VMEM((2, _FBLK, 128), jnp.int32),
            pltpu.VMEM((_REM, 128), jnp.int32),
            pltpu.VMEM((_N_PAD,), jnp.float32),
            pltpu.SemaphoreType.DMA,
        ],
        compiler_params=_params,
    )
    def hist_kernel(src_hbm, out_hbm, idx_v, rem_v, hist_v, sem):
        c = lax.axis_index("c")
        s = lax.axis_index("s")
        wid = c * _NS + s

        zeros16 = jnp.zeros((16,), jnp.float32)
        ones16 = jnp.ones((16,), jnp.float32)

        @plsc.parallel_loop(0, _N_PAD // 16, unroll=16)
        def zfill(i):
            hist_v[pl.ds(pl.multiple_of(i * 16, 16), 16)] = zeros16

        tile_base = wid * _BLK

        def start(f):
            off = pl.multiple_of(tile_base + f * _FBLK, 1)
            return pltpu.async_copy(
                src_hbm.at[pl.ds(off, _FBLK), 0], idx_v.at[f % 2], sem
            )

        cp = start(0)
        for f in range(_FETCHES):
            cp.wait()
            if f + 1 < _FETCHES:
                cp = start(f + 1)
            buf = idx_v.at[f % 2]

            @plsc.parallel_loop(0, _FBLK, unroll=_UNROLL)
            def group(b):
                for u in range(8):
                    idx = buf[b, pl.ds(pl.multiple_of(u * 16, 16), 16)]
                    plsc.addupdate_scatter(hist_v, [idx], ones16)

        @pl.when(wid == 0)
        def _():
            pltpu.sync_copy(src_hbm.at[pl.ds(_NW * _BLK, _REM), 0], rem_v)

            @plsc.parallel_loop(0, _REM, unroll=_UNROLL)
            def rem_group(b):
                for u in range(8):
                    idx = rem_v[b, pl.ds(pl.multiple_of(u * 16, 16), 16)]
                    plsc.addupdate_scatter(hist_v, [idx], ones16)

        pltpu.sync_copy(hist_v, out_hbm.at[wid])

    return hist_kernel(src)


def _sc_combine(partials, n_arr):
    @functools.partial(
        pl.kernel,
        mesh=_mesh,
        out_type=jax.ShapeDtypeStruct((_N_PAD,), jnp.float32),
        scratch_types=[
            pltpu.VMEM((_NW, _STRIP), jnp.float32),
            pltpu.VMEM((_STRIP,), jnp.float32),
            pltpu.VMEM((16,), jnp.int32),
            pltpu.SemaphoreType.DMA,
        ],
        compiler_params=_params,
    )
    def combine_kernel(parts_hbm, n_hbm, out_hbm, stage_v, acc_v, n_v, sem):
        c = lax.axis_index("c")
        s = lax.axis_index("s")
        wid = c * _NS + s
        base = pl.multiple_of(wid * _STRIP, 8)

        pltpu.sync_copy(n_hbm, n_v)
        cps = [
            pltpu.async_copy(
                parts_hbm.at[r, pl.ds(base, _STRIP)], stage_v.at[r], sem
            )
            for r in range(_NW)
        ]
        for cp in cps:
            cp.wait()

        n_vec = n_v[pl.ds(0, 16)]
        lanes = lax.iota(jnp.int32, 16)

        @plsc.parallel_loop(0, _STRIP // 16, unroll=7)
        def group(g):
            off = pl.multiple_of(g * 16, 16)
            acc = stage_v[0, pl.ds(off, 16)]
            for r in range(1, _NW):
                acc = acc + stage_v[r, pl.ds(off, 16)]
            bin_ix = (wid * _STRIP + off) + lanes
            acc_v[pl.ds(off, 16)] = jnp.where(bin_ix < n_vec, acc, 0.0)

        pltpu.sync_copy(acc_v, out_hbm.at[pl.ds(base, _STRIP)])

    return combine_kernel(partials, n_arr)


def kernel(edge_index, num_nodes):
    view = jnp.transpose(
        edge_index.astype(jnp.int32).reshape(2, _NBLOCKS, 128), (1, 0, 2)
    )
    partials = _sc_histogram(view)
    n_arr = jnp.full((16,), num_nodes, jnp.int32)
    deg = _sc_combine(partials, n_arr)
    return deg[:_N_NODES].reshape(_N_NODES, 1)
